# baseline (speedup 1.0000x reference)
import functools

import jax
import jax.numpy as jnp
from jax import lax
from jax.experimental import pallas as pl
from jax.experimental.pallas import tpu as pltpu
from jax.experimental.pallas import tpu_sc as plsc

VOCAB = 1000
VPAD = 1024
NTILE = VPAD // 128
B, T = 1024, 200
NSPLIT = 8
BK = B // NSPLIT
NC, NS = 2, 16
NW = NC * NS
B_W = BK // NW
CB, CT = 4, 8
CHUNK = CB * CT
NTG = T // CT
PER_W = B_W * T
NCHUNK = PER_W // CHUNK
NVR = CHUNK // 16
NK = BK * T

_mesh = plsc.VectorSubcoreMesh(core_axis_name="c", subcore_axis_name="s")


@functools.partial(
    pl.kernel,
    out_type=jax.ShapeDtypeStruct((BK, T, VOCAB), jnp.float32),
    mesh=_mesh,
    compiler_params=pltpu.CompilerParams(needs_layout_passes=False),
    scratch_types=[
        pltpu.VMEM((PER_W,), jnp.int32),
        pltpu.VMEM((3, NTILE, CHUNK), jnp.int32),
        pltpu.VMEM((CB, CT, VOCAB), jnp.float32),
        pltpu.VMEM((CB, CT, VOCAB), jnp.float32),
        pltpu.VMEM((CB, CT, VOCAB), jnp.float32),
        pltpu.VMEM((3, CB, CT, 128), jnp.float32),
        pltpu.SemaphoreType.DMA,
        pltpu.SemaphoreType.DMA,
        pltpu.SemaphoreType.DMA,
        pltpu.SemaphoreType.DMA,
        pltpu.SemaphoreType.DMA,
        pltpu.SemaphoreType.DMA,
    ],
)
def _gather_rows(idx_hbm, table_hbm, out_hbm, idx_v, ibuf, ob0, ob1, ob2,
                 tbuf, gsem0, gsem1, gsem2, osem0, osem1, osem2):
    wid = lax.axis_index("s") * NC + lax.axis_index("c")
    base = wid * PER_W
    b_base = wid * B_W
    pltpu.sync_copy(idx_hbm.at[pl.ds(base, PER_W)], idx_v)

    obufs = (ob0, ob1, ob2)
    gsems = (gsem0, gsem1, gsem2)
    osems = (osem0, osem1, osem2)

    def gathers(i, b):
        cps = []
        for tc in range(NTILE - 1):
            for bl in range(CB):
                cps.append(pltpu.make_async_copy(
                    table_hbm.at[ibuf.at[b, tc, pl.ds(CT * bl, CT)]],
                    obufs[b].at[bl].at[:, pl.ds(128 * tc, 128)], gsems[b]))
        for bl in range(CB):
            cps.append(pltpu.make_async_copy(
                table_hbm.at[ibuf.at[b, NTILE - 1, pl.ds(CT * bl, CT)]],
                tbuf.at[b, bl], gsems[b]))
        return cps

    def outcopy(i, b):
        b0 = b_base + CB * (i // NTG)
        t0 = CT * (i % NTG)
        return pltpu.make_async_copy(
            obufs[b], out_hbm.at[pl.ds(b0, CB), pl.ds(t0, CT)], osems[b])

    def issue(i, b):
        vs = [idx_v[pl.ds(i * CHUNK + 16 * k, 16)] for k in range(NVR)]
        for tc in range(NTILE):
            for k in range(NVR):
                ibuf[b, tc, pl.ds(16 * k, 16)] = vs[k] + 8 * tc
        for cp in gathers(i, b):
            cp.start()

    lanes = lax.iota(jnp.int32, 16)
    tail_cols = 992 + lanes
    tail_mask = lanes < 8

    def drain(i, b):
        for cp in gathers(i, b):
            cp.wait()

        def row_move(r, carry):
            bl = r // CT
            rr = r % CT
            for k in range(6):
                obufs[b][bl, rr, pl.ds(896 + 16 * k, 16)] = \
                    tbuf[b, bl, rr, pl.ds(16 * k, 16)]
            x = tbuf[b, bl, rr, pl.ds(96, 16)]
            blv = jnp.full((16,), bl, jnp.int32)
            rvec = jnp.full((16,), rr, jnp.int32)
            plsc.store_scatter(obufs[b], [blv, rvec, tail_cols], x,
                               mask=tail_mask)
            return carry

        lax.fori_loop(0, CHUNK, row_move, 0)
        outcopy(i, b).start()

    for b in range(3):
        issue(b, b)
    drain(0, 0)

    def body(p, carry):
        for b in range(3):
            i = 3 * p + b + 1
            bg = (b + 1) % 3

            @pl.when(i + 2 < NCHUNK)
            def _():
                outcopy(i - 1, b).wait()
                issue(i + 2, b)

            @pl.when(i < NCHUNK)
            def _():
                drain(i, bg)
        return carry

    lax.fori_loop(0, (NCHUNK + 1) // 3, body, 0)

    for j in range(NCHUNK - 3, NCHUNK):
        outcopy(j, j % 3).wait()


def kernel(idx, token_embedding_table):
    pad = jnp.pad(token_embedding_table, ((0, 0), (0, VPAD - VOCAB)))
    table_r = pad.reshape(VOCAB // 8, 8, NTILE, 128).transpose(0, 2, 1, 3)
    table_r = table_r.reshape(VOCAB * NTILE, 128)
    idx_p = idx.astype(jnp.int32).reshape(NSPLIT, NW, B_W // CB, CB, NTG, CT)
    idx_p = idx_p.transpose(0, 1, 2, 4, 3, 5).reshape(NSPLIT, NK)
    base_idx = (idx_p // 8) * 64 + idx_p % 8
    parts = [_gather_rows(base_idx[k], table_r) for k in range(NSPLIT)]
    return jnp.concatenate(parts, axis=0)


# speedup vs baseline: 1.4984x; 1.4984x over previous
import functools

import jax
import jax.numpy as jnp
from jax import lax
from jax.experimental import pallas as pl
from jax.experimental.pallas import tpu as pltpu
from jax.experimental.pallas import tpu_sc as plsc

VOCAB = 1000
VPAD = 1024
NTILE = VPAD // 128
B, T = 1024, 200
N = B * T
NC, NS = 2, 16
NW = NC * NS
PER_W = N // NW
CHUNK = 32
NCHUNK = PER_W // CHUNK
NVR = CHUNK // 16

_mesh = plsc.VectorSubcoreMesh(core_axis_name="c", subcore_axis_name="s")


@functools.partial(
    pl.kernel,
    out_type=jax.ShapeDtypeStruct((N, VOCAB), jnp.float32),
    mesh=_mesh,
    compiler_params=pltpu.CompilerParams(needs_layout_passes=False),
    scratch_types=[
        pltpu.VMEM((PER_W,), jnp.int32),
        pltpu.VMEM((3, NTILE, CHUNK), jnp.int32),
        pltpu.VMEM((CHUNK, VOCAB), jnp.float32),
        pltpu.VMEM((CHUNK, VOCAB), jnp.float32),
        pltpu.VMEM((CHUNK, VOCAB), jnp.float32),
        pltpu.VMEM((3, CHUNK, 128), jnp.float32),
        pltpu.SemaphoreType.DMA,
        pltpu.SemaphoreType.DMA,
        pltpu.SemaphoreType.DMA,
        pltpu.SemaphoreType.DMA,
        pltpu.SemaphoreType.DMA,
        pltpu.SemaphoreType.DMA,
    ],
)
def _gather_rows(idx_hbm, table_hbm, out_hbm, idx_v, ibuf, ob0, ob1, ob2,
                 tbuf, gsem0, gsem1, gsem2, osem0, osem1, osem2):
    wid = lax.axis_index("s") * NC + lax.axis_index("c")
    base = wid * PER_W
    pltpu.sync_copy(idx_hbm.at[pl.ds(base, PER_W)], idx_v)

    obufs = (ob0, ob1, ob2)
    gsems = (gsem0, gsem1, gsem2)
    osems = (osem0, osem1, osem2)

    def gathers(i, b):
        cps = []
        for tc in range(NTILE - 1):
            cps.append(pltpu.make_async_copy(
                table_hbm.at[ibuf.at[b, tc]],
                obufs[b].at[:, pl.ds(128 * tc, 128)], gsems[b]))
        cps.append(pltpu.make_async_copy(
            table_hbm.at[ibuf.at[b, NTILE - 1]], tbuf.at[b], gsems[b]))
        return cps

    def outcopy(i, b):
        return pltpu.make_async_copy(
            obufs[b], out_hbm.at[pl.ds(base + i * CHUNK, CHUNK)], osems[b])

    def issue(i, b):
        vs = [idx_v[pl.ds(i * CHUNK + 16 * k, 16)] for k in range(NVR)]
        for tc in range(NTILE):
            for k in range(NVR):
                ibuf[b, tc, pl.ds(16 * k, 16)] = vs[k] + 8 * tc
        for cp in gathers(i, b):
            cp.start()

    lanes = lax.iota(jnp.int32, 16)
    tail_cols = 992 + lanes
    tail_mask = lanes < 8

    def drain(i, b):
        for cp in gathers(i, b):
            cp.wait()

        def row_move(r, carry):
            for k in range(6):
                obufs[b][r, pl.ds(896 + 16 * k, 16)] = tbuf[b, r, pl.ds(16 * k, 16)]
            x = tbuf[b, r, pl.ds(96, 16)]
            rvec = jnp.full((16,), r, jnp.int32)
            plsc.store_scatter(obufs[b], [rvec, tail_cols], x, mask=tail_mask)
            return carry

        lax.fori_loop(0, CHUNK, row_move, 0)
        outcopy(i, b).start()

    for b in range(3):
        issue(b, b)
    drain(0, 0)

    def body(p, carry):
        for b in range(3):
            i = 3 * p + b + 1
            bg = (b + 1) % 3

            @pl.when(i + 2 < NCHUNK)
            def _():
                outcopy(i - 1, b).wait()
                issue(i + 2, b)

            @pl.when(i < NCHUNK)
            def _():
                drain(i, bg)
        return carry

    lax.fori_loop(0, (NCHUNK + 1) // 3, body, 0)

    for j in range(NCHUNK - 3, NCHUNK):
        outcopy(j, j % 3).wait()


def kernel(idx, token_embedding_table):
    pad = jnp.pad(token_embedding_table, ((0, 0), (0, VPAD - VOCAB)))
    table_r = pad.reshape(VOCAB // 8, 8, NTILE, 128).transpose(0, 2, 1, 3)
    table_r = table_r.reshape(VOCAB * NTILE, 128)
    flat_idx = idx.reshape(N).astype(jnp.int32)
    base_idx = (flat_idx // 8) * 64 + flat_idx % 8
    flat = _gather_rows(base_idx, table_r)
    return flat.reshape(B, T, VOCAB)
